# baseline (speedup 1.0000x reference)
import jax
import jax.numpy as jnp
import numpy as np
from jax.experimental import pallas as pl
from jax.experimental.pallas import tpu as pltpu

B, S, D, H, L, DFF, E = 1, 2048, 1024, 16, 2, 2048, 8
DH = D // H
V = 90
PAD_IDX = 2
RT = 256
NRT = S // RT
M = 128
NBUF = 5120
NT = NBUF // M
F32 = jnp.float32


def _ln_in(x, g, b):
    mu = jnp.mean(x, axis=-1, keepdims=True)
    var = jnp.mean((x - mu) ** 2, axis=-1, keepdims=True)
    return (x - mu) / jnp.sqrt(var + 1e-5) * g + b


def _embed_k(tok_ref, emb_ref, pe_ref, out_ref):
    tok = tok_ref[...]
    oh = (tok == jax.lax.broadcasted_iota(jnp.int32, (RT, V), 1)).astype(F32)
    out_ref[...] = jnp.dot(oh, emb_ref[...], preferred_element_type=F32) + pe_ref[...]


def _embed(tok2d, emb, pe):
    return pl.pallas_call(
        _embed_k,
        grid=(NRT,),
        in_specs=[
            pl.BlockSpec((RT, 1), lambda i: (i, 0)),
            pl.BlockSpec((V, D), lambda i: (0, 0)),
            pl.BlockSpec((RT, D), lambda i: (i, 0)),
        ],
        out_specs=pl.BlockSpec((RT, D), lambda i: (i, 0)),
        out_shape=jax.ShapeDtypeStruct((S, D), F32),
    )(tok2d, emb, pe)


def _mm_bias_k(x_ref, w_ref, b_ref, o_ref):
    o_ref[...] = jnp.dot(x_ref[...], w_ref[...], preferred_element_type=F32) + b_ref[...]


def _mm_bias(x, w, b):
    n = w.shape[1]
    return pl.pallas_call(
        _mm_bias_k,
        grid=(NRT,),
        in_specs=[
            pl.BlockSpec((RT, D), lambda i: (i, 0)),
            pl.BlockSpec((D, n), lambda i: (0, 0)),
            pl.BlockSpec((1, n), lambda i: (0, 0)),
        ],
        out_specs=pl.BlockSpec((RT, n), lambda i: (i, 0)),
        out_shape=jax.ShapeDtypeStruct((S, n), F32),
    )(x, w, b.reshape(1, n))


def _attn_k(tok_ref, q_ref, k_ref, v_ref, o_ref):
    q = q_ref[0]
    k = k_ref[0]
    v = v_ref[0]
    s = jax.lax.dot_general(q, k, (((1,), (1,)), ((), ())),
                            preferred_element_type=F32) * (1.0 / 8.0)
    pad = tok_ref[...] == PAD_IDX
    s = jnp.where(pad, -1e9, s)
    m = jnp.max(s, axis=-1, keepdims=True)
    p = jnp.exp(s - m)
    p = p / jnp.sum(p, axis=-1, keepdims=True)
    o_ref[0] = jnp.dot(p, v, preferred_element_type=F32)


def _attention(tok_row, q, k, v):
    return pl.pallas_call(
        _attn_k,
        grid=(H,),
        in_specs=[
            pl.BlockSpec((1, S), lambda h: (0, 0)),
            pl.BlockSpec((1, S, DH), lambda h: (h, 0, 0)),
            pl.BlockSpec((1, S, DH), lambda h: (h, 0, 0)),
            pl.BlockSpec((1, S, DH), lambda h: (h, 0, 0)),
        ],
        out_specs=pl.BlockSpec((1, S, DH), lambda h: (h, 0, 0)),
        out_shape=jax.ShapeDtypeStruct((H, S, DH), F32),
    )(tok_row, q, k, v)


def _proj_ln_k(o_ref, x_ref, w_ref, b_ref, g_ref, bn_ref, h_ref):
    t = jnp.dot(o_ref[...], w_ref[...], preferred_element_type=F32)
    t = t + b_ref[...] + x_ref[...]
    h_ref[...] = _ln_in(t, g_ref[...], bn_ref[...])


def _proj_ln(o, x, w, b, g, bn):
    return pl.pallas_call(
        _proj_ln_k,
        grid=(NRT,),
        in_specs=[
            pl.BlockSpec((RT, D), lambda i: (i, 0)),
            pl.BlockSpec((RT, D), lambda i: (i, 0)),
            pl.BlockSpec((D, D), lambda i: (0, 0)),
            pl.BlockSpec((1, D), lambda i: (0, 0)),
            pl.BlockSpec((1, D), lambda i: (0, 0)),
            pl.BlockSpec((1, D), lambda i: (0, 0)),
        ],
        out_specs=pl.BlockSpec((RT, D), lambda i: (i, 0)),
        out_shape=jax.ShapeDtypeStruct((S, D), F32),
    )(o, x, w, b.reshape(1, D), g.reshape(1, D), bn.reshape(1, D))


def _router_k(h_ref, wr_ref, br_ref, rl_ref, tv_ref, ti_ref, ent_ref):
    rl = jnp.dot(h_ref[...], wr_ref[...], preferred_element_type=F32) + br_ref[...]
    rl_ref[...] = rl
    mx = jnp.max(rl, axis=-1, keepdims=True)
    ex = jnp.exp(rl - mx)
    probs = ex / jnp.sum(ex, axis=-1, keepdims=True)
    ent = jnp.mean(-jnp.sum(probs * jnp.log(probs + 1e-9), axis=-1))
    ent_ref[...] = ent.reshape(1, 1)
    lane = jax.lax.broadcasted_iota(jnp.int32, (S, E), 1)
    p1 = jnp.max(probs, axis=-1, keepdims=True)
    i1 = jnp.min(jnp.where(probs == p1, lane, E), axis=-1, keepdims=True)
    pm = jnp.where(lane == i1, -1.0, probs)
    p2 = jnp.max(pm, axis=-1, keepdims=True)
    i2 = jnp.min(jnp.where(pm == p2, lane, E), axis=-1, keepdims=True)
    dn = p1 + p2 + 1e-9
    tv_ref[...] = jnp.concatenate([p1 / dn, p2 / dn], axis=-1)
    ti_ref[...] = jnp.concatenate([i1, i2], axis=-1)


def _router(h, wr, br):
    return pl.pallas_call(
        _router_k,
        grid=(1,),
        in_specs=[
            pl.BlockSpec((S, D), lambda i: (0, 0)),
            pl.BlockSpec((D, E), lambda i: (0, 0)),
            pl.BlockSpec((1, E), lambda i: (0, 0)),
        ],
        out_specs=[
            pl.BlockSpec((S, E), lambda i: (0, 0)),
            pl.BlockSpec((S, 2), lambda i: (0, 0)),
            pl.BlockSpec((S, 2), lambda i: (0, 0)),
            pl.BlockSpec((1, 1), lambda i: (0, 0)),
        ],
        out_shape=[
            jax.ShapeDtypeStruct((S, E), F32),
            jax.ShapeDtypeStruct((S, 2), F32),
            jax.ShapeDtypeStruct((S, 2), jnp.int32),
            jax.ShapeDtypeStruct((1, 1), F32),
        ],
    )(h, wr, br.reshape(1, E))


def _moe_k(ept_ref, x_ref, w1_ref, b1_ref, w2_ref, b2_ref, o_ref):
    hid = jnp.dot(x_ref[...], w1_ref[0], preferred_element_type=F32) + b1_ref[0]
    hid = jnp.maximum(hid, 0.0)
    o_ref[...] = jnp.dot(hid, w2_ref[0], preferred_element_type=F32) + b2_ref[0]


def _moe_grouped(xg, w1, b1, w2, b2, ept):
    grid_spec = pltpu.PrefetchScalarGridSpec(
        num_scalar_prefetch=1,
        grid=(NT,),
        in_specs=[
            pl.BlockSpec((M, D), lambda i, ept: (i, 0)),
            pl.BlockSpec((1, D, DFF), lambda i, ept: (ept[i], 0, 0)),
            pl.BlockSpec((1, 1, DFF), lambda i, ept: (ept[i], 0, 0)),
            pl.BlockSpec((1, DFF, D), lambda i, ept: (ept[i], 0, 0)),
            pl.BlockSpec((1, 1, D), lambda i, ept: (ept[i], 0, 0)),
        ],
        out_specs=pl.BlockSpec((M, D), lambda i, ept: (i, 0)),
    )
    return pl.pallas_call(
        _moe_k,
        grid_spec=grid_spec,
        out_shape=jax.ShapeDtypeStruct((NBUF, D), F32),
    )(ept, xg, w1, b1.reshape(E, 1, DFF), w2, b2.reshape(E, 1, D))


def _combine_k(h_ref, e0_ref, e1_ref, t0_ref, t1_ref, g_ref, b_ref, x_ref):
    moe = t0_ref[...] * e0_ref[...] + t1_ref[...] * e1_ref[...]
    x_ref[...] = _ln_in(h_ref[...] + moe, g_ref[...], b_ref[...])


def _combine(h, e0, e1, t0, t1, g, bn):
    return pl.pallas_call(
        _combine_k,
        grid=(NRT,),
        in_specs=[
            pl.BlockSpec((RT, D), lambda i: (i, 0)),
            pl.BlockSpec((RT, D), lambda i: (i, 0)),
            pl.BlockSpec((RT, D), lambda i: (i, 0)),
            pl.BlockSpec((RT, 1), lambda i: (i, 0)),
            pl.BlockSpec((RT, 1), lambda i: (i, 0)),
            pl.BlockSpec((1, D), lambda i: (0, 0)),
            pl.BlockSpec((1, D), lambda i: (0, 0)),
        ],
        out_specs=pl.BlockSpec((RT, D), lambda i: (i, 0)),
        out_shape=jax.ShapeDtypeStruct((S, D), F32),
    )(h, e0, e1, t0, t1, g.reshape(1, D), bn.reshape(1, D))


def _head_k(x_ref, wa_ref, ba_ref, g_ref, b_ref, wb_ref, bb_ref, tok_ref,
            mt_ref, out_ref):
    y = jnp.dot(x_ref[...], wa_ref[...], preferred_element_type=F32) + ba_ref[...]
    y = _ln_in(y, g_ref[...], b_ref[...])
    y = jnp.maximum(y, 0.0)
    lg = jnp.dot(y, wb_ref[...], preferred_element_type=F32) + bb_ref[...]
    oh = (tok_ref[...] == jax.lax.broadcasted_iota(jnp.int32, (RT, V), 1)).astype(F32)
    am = jnp.dot(oh, mt_ref[...], preferred_element_type=F32)
    out_ref[...] = jnp.where(am > 0.5, lg, -60000.0)


def _head(x, wa, ba, gh, bh, wb, bb, tok2d, mt):
    D2 = D // 2
    return pl.pallas_call(
        _head_k,
        grid=(NRT,),
        in_specs=[
            pl.BlockSpec((RT, D), lambda i: (i, 0)),
            pl.BlockSpec((D, D2), lambda i: (0, 0)),
            pl.BlockSpec((1, D2), lambda i: (0, 0)),
            pl.BlockSpec((1, D2), lambda i: (0, 0)),
            pl.BlockSpec((1, D2), lambda i: (0, 0)),
            pl.BlockSpec((D2, V), lambda i: (0, 0)),
            pl.BlockSpec((1, V), lambda i: (0, 0)),
            pl.BlockSpec((RT, 1), lambda i: (i, 0)),
            pl.BlockSpec((V, V), lambda i: (0, 0)),
        ],
        out_specs=pl.BlockSpec((RT, V), lambda i: (i, 0)),
        out_shape=jax.ShapeDtypeStruct((S, V), F32),
    )(x, wa, ba.reshape(1, D2), gh.reshape(1, D2), bh.reshape(1, D2),
      wb, bb.reshape(1, V), tok2d, mt)


def _route_metadata(ti):
    fi = (ti[:, :, None] == jnp.arange(E, dtype=jnp.int32)[None, None, :])
    fi = fi.any(axis=1).astype(jnp.int32)
    pos_incl = jnp.cumsum(fi, axis=0)
    counts = pos_incl[-1]
    pos = pos_incl - fi
    ru = ((counts + M - 1) // M) * M
    off = jnp.concatenate([jnp.zeros((1,), jnp.int32),
                           jnp.cumsum(ru)[:-1].astype(jnp.int32)])
    dst = off[ti] + jnp.take_along_axis(pos, ti, axis=1)
    tokid = jnp.broadcast_to(jnp.arange(S, dtype=jnp.int32)[:, None], (S, 2))
    tok_per_slot = jnp.zeros((NBUF,), jnp.int32).at[dst.reshape(-1)].set(
        tokid.reshape(-1), mode="drop")
    tile_starts = jnp.arange(NT, dtype=jnp.int32) * M
    ept = (jnp.searchsorted(off, tile_starts, side="right") - 1).astype(jnp.int32)
    return dst, tok_per_slot, ept


def kernel(tokenizer_encoded_proteins, mask_table, params):
    p = params
    tok = tokenizer_encoded_proteins.reshape(S).astype(jnp.int32)
    tok2d = tok.reshape(S, 1)
    tok_row = tok.reshape(1, S)
    mt_f = mask_table.astype(F32)

    pos = jnp.arange(S, dtype=F32)[:, None]
    i = jnp.arange(D // 2, dtype=F32)[None, :]
    angle = pos / jnp.power(10000.0, (2.0 * i) / D)
    pe = jnp.concatenate([jnp.sin(angle), jnp.cos(angle)], axis=-1)

    x = _embed(tok2d, p['emb'], pe)

    router_list = []
    ent = []
    for l in range(L):
        wqkv = jnp.concatenate([p['Wq'][l], p['Wk'][l], p['Wv'][l]], axis=1)
        bqkv = jnp.concatenate([p['bq'][l], p['bk'][l], p['bv'][l]], axis=0)
        qkv = _mm_bias(x, wqkv, bqkv)
        q = qkv[:, :D].reshape(S, H, DH).transpose(1, 0, 2)
        k = qkv[:, D:2 * D].reshape(S, H, DH).transpose(1, 0, 2)
        v = qkv[:, 2 * D:].reshape(S, H, DH).transpose(1, 0, 2)
        o = _attention(tok_row, q, k, v)
        o = o.transpose(1, 0, 2).reshape(S, D)
        h = _proj_ln(o, x, p['Wo'][l], p['bo'][l], p['g1'][l], p['b1n'][l])
        rl, tv, ti, e = _router(h, p['Wr'][l], p['br'][l])
        router_list.append(rl)
        ent.append(e[0, 0])
        dst, tok_per_slot, ept = _route_metadata(ti)
        xg = h[tok_per_slot]
        eo = _moe_grouped(xg, p['W1e'][l], p['b1e'][l], p['W2e'][l],
                          p['b2e'][l], ept)
        e0 = eo[dst[:, 0]]
        e1 = eo[dst[:, 1]]
        x = _combine(h, e0, e1, tv[:, 0:1], tv[:, 1:2], p['g2'][l], p['b2n'][l])

    entropy_loss = (ent[0] + ent[1]) / L
    logits = _head(x, p['Wa'], p['ba'], p['gh'], p['bh'], p['Wb'], p['bb'],
                   tok2d, mt_f).reshape(B, S, V)
    return (logits, router_list[0], router_list[1], entropy_loss)


# speedup vs baseline: 1.0414x; 1.0414x over previous
import jax
import jax.numpy as jnp
import numpy as np
from jax.experimental import pallas as pl
from jax.experimental.pallas import tpu as pltpu

B, S, D, H, L, DFF, E = 1, 2048, 1024, 16, 2, 2048, 8
DH = D // H
V = 90
PAD_IDX = 2
RT = 256
NRT = S // RT
M = 256
NBUF = 6144
NT = NBUF // M
F32 = jnp.float32
BF16 = jnp.bfloat16


def _ln_in(x, g, b):
    mu = jnp.mean(x, axis=-1, keepdims=True)
    var = jnp.mean((x - mu) ** 2, axis=-1, keepdims=True)
    return (x - mu) / jnp.sqrt(var + 1e-5) * g + b


def _embed_k(tok_ref, emb_ref, pe_ref, out_ref):
    tok = tok_ref[...]
    oh = (tok == jax.lax.broadcasted_iota(jnp.int32, (RT, V), 1)).astype(F32)
    out_ref[...] = jnp.dot(oh, emb_ref[...], preferred_element_type=F32) + pe_ref[...]


def _embed(tok2d, emb, pe):
    return pl.pallas_call(
        _embed_k,
        grid=(NRT,),
        in_specs=[
            pl.BlockSpec((RT, 1), lambda i: (i, 0)),
            pl.BlockSpec((V, D), lambda i: (0, 0)),
            pl.BlockSpec((RT, D), lambda i: (i, 0)),
        ],
        out_specs=pl.BlockSpec((RT, D), lambda i: (i, 0)),
        out_shape=jax.ShapeDtypeStruct((S, D), F32),
    )(tok2d, emb, pe)


def _mm_bias_k(x_ref, w_ref, b_ref, o_ref):
    o_ref[...] = jnp.dot(x_ref[...].astype(BF16), w_ref[...],
                         preferred_element_type=F32) + b_ref[...]


def _mm_bias(x, w, b):
    n = w.shape[1]
    return pl.pallas_call(
        _mm_bias_k,
        grid=(NRT,),
        in_specs=[
            pl.BlockSpec((RT, D), lambda i: (i, 0)),
            pl.BlockSpec((D, n), lambda i: (0, 0)),
            pl.BlockSpec((1, n), lambda i: (0, 0)),
        ],
        out_specs=pl.BlockSpec((RT, n), lambda i: (i, 0)),
        out_shape=jax.ShapeDtypeStruct((S, n), F32),
    )(x, w, b.reshape(1, n))


def _attn_k(tok_ref, q_ref, k_ref, v_ref, o_ref):
    q = q_ref[0].astype(BF16)
    k = k_ref[0].astype(BF16)
    v = v_ref[0].astype(BF16)
    s = jax.lax.dot_general(q, k, (((1,), (1,)), ((), ())),
                            preferred_element_type=F32) * (1.0 / 8.0)
    pad = tok_ref[...] == PAD_IDX
    s = jnp.where(pad, -1e9, s)
    m = jnp.max(s, axis=-1, keepdims=True)
    p = jnp.exp(s - m)
    p = p / jnp.sum(p, axis=-1, keepdims=True)
    o_ref[0] = jnp.dot(p.astype(BF16), v, preferred_element_type=F32)


def _attention(tok_row, q, k, v):
    return pl.pallas_call(
        _attn_k,
        grid=(H,),
        in_specs=[
            pl.BlockSpec((1, S), lambda h: (0, 0)),
            pl.BlockSpec((1, S, DH), lambda h: (h, 0, 0)),
            pl.BlockSpec((1, S, DH), lambda h: (h, 0, 0)),
            pl.BlockSpec((1, S, DH), lambda h: (h, 0, 0)),
        ],
        out_specs=pl.BlockSpec((1, S, DH), lambda h: (h, 0, 0)),
        out_shape=jax.ShapeDtypeStruct((H, S, DH), F32),
    )(tok_row, q, k, v)


def _proj_ln_k(o_ref, x_ref, w_ref, b_ref, g_ref, bn_ref, h_ref):
    t = jnp.dot(o_ref[...].astype(BF16), w_ref[...], preferred_element_type=F32)
    t = t + b_ref[...] + x_ref[...]
    h_ref[...] = _ln_in(t, g_ref[...], bn_ref[...])


def _proj_ln(o, x, w, b, g, bn):
    return pl.pallas_call(
        _proj_ln_k,
        grid=(NRT,),
        in_specs=[
            pl.BlockSpec((RT, D), lambda i: (i, 0)),
            pl.BlockSpec((RT, D), lambda i: (i, 0)),
            pl.BlockSpec((D, D), lambda i: (0, 0)),
            pl.BlockSpec((1, D), lambda i: (0, 0)),
            pl.BlockSpec((1, D), lambda i: (0, 0)),
            pl.BlockSpec((1, D), lambda i: (0, 0)),
        ],
        out_specs=pl.BlockSpec((RT, D), lambda i: (i, 0)),
        out_shape=jax.ShapeDtypeStruct((S, D), F32),
    )(o, x, w, b.reshape(1, D), g.reshape(1, D), bn.reshape(1, D))


def _router_k(h_ref, wr_ref, br_ref, rl_ref, tv_ref, ti_ref, ent_ref):
    rl = jnp.dot(h_ref[...], wr_ref[...], preferred_element_type=F32) + br_ref[...]
    rl_ref[...] = rl
    mx = jnp.max(rl, axis=-1, keepdims=True)
    ex = jnp.exp(rl - mx)
    probs = ex / jnp.sum(ex, axis=-1, keepdims=True)
    ent = jnp.mean(-jnp.sum(probs * jnp.log(probs + 1e-9), axis=-1))
    ent_ref[...] = ent.reshape(1, 1)
    lane = jax.lax.broadcasted_iota(jnp.int32, (S, E), 1)
    p1 = jnp.max(probs, axis=-1, keepdims=True)
    i1 = jnp.min(jnp.where(probs == p1, lane, E), axis=-1, keepdims=True)
    pm = jnp.where(lane == i1, -1.0, probs)
    p2 = jnp.max(pm, axis=-1, keepdims=True)
    i2 = jnp.min(jnp.where(pm == p2, lane, E), axis=-1, keepdims=True)
    dn = p1 + p2 + 1e-9
    tv_ref[...] = jnp.concatenate([p1 / dn, p2 / dn], axis=-1)
    ti_ref[...] = jnp.concatenate([i1, i2], axis=-1)


def _router(h, wr, br):
    return pl.pallas_call(
        _router_k,
        grid=(1,),
        in_specs=[
            pl.BlockSpec((S, D), lambda i: (0, 0)),
            pl.BlockSpec((D, E), lambda i: (0, 0)),
            pl.BlockSpec((1, E), lambda i: (0, 0)),
        ],
        out_specs=[
            pl.BlockSpec((S, E), lambda i: (0, 0)),
            pl.BlockSpec((S, 2), lambda i: (0, 0)),
            pl.BlockSpec((S, 2), lambda i: (0, 0)),
            pl.BlockSpec((1, 1), lambda i: (0, 0)),
        ],
        out_shape=[
            jax.ShapeDtypeStruct((S, E), F32),
            jax.ShapeDtypeStruct((S, 2), F32),
            jax.ShapeDtypeStruct((S, 2), jnp.int32),
            jax.ShapeDtypeStruct((1, 1), F32),
        ],
    )(h, wr, br.reshape(1, E))


def _moe_k(ept_ref, x_ref, w1_ref, b1_ref, w2_ref, b2_ref, o_ref):
    hid = jnp.dot(x_ref[...], w1_ref[0], preferred_element_type=F32) + b1_ref[0]
    hid = jnp.maximum(hid, 0.0)
    o_ref[...] = jnp.dot(hid.astype(BF16), w2_ref[0],
                         preferred_element_type=F32) + b2_ref[0]


def _moe_grouped(xg, w1, b1, w2, b2, ept):
    grid_spec = pltpu.PrefetchScalarGridSpec(
        num_scalar_prefetch=1,
        grid=(NT,),
        in_specs=[
            pl.BlockSpec((M, D), lambda i, ept: (i, 0)),
            pl.BlockSpec((1, D, DFF), lambda i, ept: (ept[i], 0, 0)),
            pl.BlockSpec((1, 1, DFF), lambda i, ept: (ept[i], 0, 0)),
            pl.BlockSpec((1, DFF, D), lambda i, ept: (ept[i], 0, 0)),
            pl.BlockSpec((1, 1, D), lambda i, ept: (ept[i], 0, 0)),
        ],
        out_specs=pl.BlockSpec((M, D), lambda i, ept: (i, 0)),
    )
    return pl.pallas_call(
        _moe_k,
        grid_spec=grid_spec,
        out_shape=jax.ShapeDtypeStruct((NBUF, D), F32),
    )(ept, xg, w1, b1.reshape(E, 1, DFF), w2, b2.reshape(E, 1, D))


def _combine_k(h_ref, e0_ref, e1_ref, t0_ref, t1_ref, g_ref, b_ref, x_ref):
    moe = t0_ref[...] * e0_ref[...] + t1_ref[...] * e1_ref[...]
    x_ref[...] = _ln_in(h_ref[...] + moe, g_ref[...], b_ref[...])


def _combine(h, e0, e1, t0, t1, g, bn):
    return pl.pallas_call(
        _combine_k,
        grid=(NRT,),
        in_specs=[
            pl.BlockSpec((RT, D), lambda i: (i, 0)),
            pl.BlockSpec((RT, D), lambda i: (i, 0)),
            pl.BlockSpec((RT, D), lambda i: (i, 0)),
            pl.BlockSpec((RT, 1), lambda i: (i, 0)),
            pl.BlockSpec((RT, 1), lambda i: (i, 0)),
            pl.BlockSpec((1, D), lambda i: (0, 0)),
            pl.BlockSpec((1, D), lambda i: (0, 0)),
        ],
        out_specs=pl.BlockSpec((RT, D), lambda i: (i, 0)),
        out_shape=jax.ShapeDtypeStruct((S, D), F32),
    )(h, e0, e1, t0, t1, g.reshape(1, D), bn.reshape(1, D))


def _head_k(x_ref, wa_ref, ba_ref, g_ref, b_ref, wb_ref, bb_ref, tok_ref,
            mt_ref, out_ref):
    y = jnp.dot(x_ref[...].astype(BF16), wa_ref[...],
                preferred_element_type=F32) + ba_ref[...]
    y = _ln_in(y, g_ref[...], b_ref[...])
    y = jnp.maximum(y, 0.0)
    lg = jnp.dot(y.astype(BF16), wb_ref[...], preferred_element_type=F32) + bb_ref[...]
    oh = (tok_ref[...] == jax.lax.broadcasted_iota(jnp.int32, (RT, V), 1)).astype(F32)
    am = jnp.dot(oh, mt_ref[...], preferred_element_type=F32)
    out_ref[...] = jnp.where(am > 0.5, lg, -60000.0)


def _head(x, wa, ba, gh, bh, wb, bb, tok2d, mt):
    D2 = D // 2
    return pl.pallas_call(
        _head_k,
        grid=(NRT,),
        in_specs=[
            pl.BlockSpec((RT, D), lambda i: (i, 0)),
            pl.BlockSpec((D, D2), lambda i: (0, 0)),
            pl.BlockSpec((1, D2), lambda i: (0, 0)),
            pl.BlockSpec((1, D2), lambda i: (0, 0)),
            pl.BlockSpec((1, D2), lambda i: (0, 0)),
            pl.BlockSpec((D2, V), lambda i: (0, 0)),
            pl.BlockSpec((1, V), lambda i: (0, 0)),
            pl.BlockSpec((RT, 1), lambda i: (i, 0)),
            pl.BlockSpec((V, V), lambda i: (0, 0)),
        ],
        out_specs=pl.BlockSpec((RT, V), lambda i: (i, 0)),
        out_shape=jax.ShapeDtypeStruct((S, V), F32),
    )(x, wa, ba.reshape(1, D2), gh.reshape(1, D2), bh.reshape(1, D2),
      wb, bb.reshape(1, V), tok2d, mt)


def _route_metadata(ti):
    fi = (ti[:, :, None] == jnp.arange(E, dtype=jnp.int32)[None, None, :])
    fi = fi.any(axis=1).astype(jnp.int32)
    pos_incl = jnp.cumsum(fi, axis=0)
    counts = pos_incl[-1]
    pos = pos_incl - fi
    ru = ((counts + M - 1) // M) * M
    off = jnp.concatenate([jnp.zeros((1,), jnp.int32),
                           jnp.cumsum(ru)[:-1].astype(jnp.int32)])
    dst = off[ti] + jnp.take_along_axis(pos, ti, axis=1)
    tokid = jnp.broadcast_to(jnp.arange(S, dtype=jnp.int32)[:, None], (S, 2))
    tok_per_slot = jnp.zeros((NBUF,), jnp.int32).at[dst.reshape(-1)].set(
        tokid.reshape(-1), mode="drop")
    tile_starts = jnp.arange(NT, dtype=jnp.int32) * M
    ept = (jnp.searchsorted(off, tile_starts, side="right") - 1).astype(jnp.int32)
    return dst, tok_per_slot, ept


def kernel(tokenizer_encoded_proteins, mask_table, params):
    p = params
    tok = tokenizer_encoded_proteins.reshape(S).astype(jnp.int32)
    tok2d = tok.reshape(S, 1)
    tok_row = tok.reshape(1, S)
    mt_f = mask_table.astype(F32)

    pos = jnp.arange(S, dtype=F32)[:, None]
    i = jnp.arange(D // 2, dtype=F32)[None, :]
    angle = pos / jnp.power(10000.0, (2.0 * i) / D)
    pe = jnp.concatenate([jnp.sin(angle), jnp.cos(angle)], axis=-1)

    x = _embed(tok2d, p['emb'], pe)

    router_list = []
    ent = []
    for l in range(L):
        wqkv = jnp.concatenate([p['Wq'][l], p['Wk'][l], p['Wv'][l]],
                               axis=1).astype(BF16)
        bqkv = jnp.concatenate([p['bq'][l], p['bk'][l], p['bv'][l]], axis=0)
        qkv = _mm_bias(x, wqkv, bqkv)
        q = qkv[:, :D].reshape(S, H, DH).transpose(1, 0, 2)
        k = qkv[:, D:2 * D].reshape(S, H, DH).transpose(1, 0, 2)
        v = qkv[:, 2 * D:].reshape(S, H, DH).transpose(1, 0, 2)
        o = _attention(tok_row, q, k, v)
        o = o.transpose(1, 0, 2).reshape(S, D)
        h = _proj_ln(o, x, p['Wo'][l].astype(BF16), p['bo'][l],
                     p['g1'][l], p['b1n'][l])
        rl, tv, ti, e = _router(h, p['Wr'][l], p['br'][l])
        router_list.append(rl)
        ent.append(e[0, 0])
        dst, tok_per_slot, ept = _route_metadata(ti)
        xg = h.astype(BF16)[tok_per_slot]
        eo = _moe_grouped(xg, p['W1e'][l].astype(BF16), p['b1e'][l],
                          p['W2e'][l].astype(BF16), p['b2e'][l], ept)
        e0 = eo[dst[:, 0]]
        e1 = eo[dst[:, 1]]
        x = _combine(h, e0, e1, tv[:, 0:1], tv[:, 1:2], p['g2'][l], p['b2n'][l])

    entropy_loss = (ent[0] + ent[1]) / L
    logits = _head(x, p['Wa'].astype(BF16), p['ba'], p['gh'], p['bh'],
                   p['Wb'].astype(BF16), p['bb'], tok2d, mt_f).reshape(B, S, V)
    return (logits, router_list[0], router_list[1], entropy_loss)


# speedup vs baseline: 1.0442x; 1.0027x over previous
import jax
import jax.numpy as jnp
import numpy as np
from jax.experimental import pallas as pl
from jax.experimental.pallas import tpu as pltpu

B, S, D, H, L, DFF, E = 1, 2048, 1024, 16, 2, 2048, 8
DH = D // H
V = 90
PAD_IDX = 2
RT = 256
NRT = S // RT
M = 256
NBUF = 6144
NT = NBUF // M
F32 = jnp.float32
BF16 = jnp.bfloat16


def _ln_in(x, g, b):
    mu = jnp.mean(x, axis=-1, keepdims=True)
    var = jnp.mean((x - mu) ** 2, axis=-1, keepdims=True)
    return (x - mu) / jnp.sqrt(var + 1e-5) * g + b


def _embed_k(tok_ref, emb_ref, pe_ref, out_ref):
    tok = tok_ref[...]
    oh = (tok == jax.lax.broadcasted_iota(jnp.int32, (RT, V), 1)).astype(F32)
    out_ref[...] = jnp.dot(oh, emb_ref[...], preferred_element_type=F32) + pe_ref[...]


def _embed(tok2d, emb, pe):
    return pl.pallas_call(
        _embed_k,
        grid=(NRT,),
        in_specs=[
            pl.BlockSpec((RT, 1), lambda i: (i, 0)),
            pl.BlockSpec((V, D), lambda i: (0, 0)),
            pl.BlockSpec((RT, D), lambda i: (i, 0)),
        ],
        out_specs=pl.BlockSpec((RT, D), lambda i: (i, 0)),
        out_shape=jax.ShapeDtypeStruct((S, D), F32),
    )(tok2d, emb, pe)


def _mm_bias_k(x_ref, w_ref, b_ref, o_ref):
    o_ref[...] = jnp.dot(x_ref[...].astype(BF16), w_ref[...],
                         preferred_element_type=F32) + b_ref[...]


def _mm_bias(x, w, b):
    n = w.shape[1]
    return pl.pallas_call(
        _mm_bias_k,
        grid=(NRT,),
        in_specs=[
            pl.BlockSpec((RT, D), lambda i: (i, 0)),
            pl.BlockSpec((D, n), lambda i: (0, 0)),
            pl.BlockSpec((1, n), lambda i: (0, 0)),
        ],
        out_specs=pl.BlockSpec((RT, n), lambda i: (i, 0)),
        out_shape=jax.ShapeDtypeStruct((S, n), F32),
    )(x, w, b.reshape(1, n))


def _attn_k(tok_ref, q_ref, k_ref, v_ref, o_ref):
    q = q_ref[0].astype(BF16)
    k = k_ref[0].astype(BF16)
    v = v_ref[0].astype(BF16)
    s = jax.lax.dot_general(q, k, (((1,), (1,)), ((), ())),
                            preferred_element_type=F32) * (1.0 / 8.0)
    pad = tok_ref[...] == PAD_IDX
    s = jnp.where(pad, -1e9, s)
    m = jnp.max(s, axis=-1, keepdims=True)
    p = jnp.exp(s - m).astype(BF16)
    r = 1.0 / jnp.sum(p.astype(F32), axis=-1, keepdims=True)
    o = jnp.dot(p, v, preferred_element_type=F32)
    o_ref[0] = o * r


def _attention(tok_row, q, k, v):
    return pl.pallas_call(
        _attn_k,
        grid=(H,),
        in_specs=[
            pl.BlockSpec((1, S), lambda h: (0, 0)),
            pl.BlockSpec((1, S, DH), lambda h: (h, 0, 0)),
            pl.BlockSpec((1, S, DH), lambda h: (h, 0, 0)),
            pl.BlockSpec((1, S, DH), lambda h: (h, 0, 0)),
        ],
        out_specs=pl.BlockSpec((1, S, DH), lambda h: (h, 0, 0)),
        out_shape=jax.ShapeDtypeStruct((H, S, DH), F32),
    )(tok_row, q, k, v)


def _proj_router_k(o_ref, x_ref, w_ref, b_ref, g_ref, bn_ref, wr_ref, br_ref,
                   h_ref, rl_ref, tv_ref, ti_ref, ent_ref):
    t = jnp.dot(o_ref[...].astype(BF16), w_ref[...], preferred_element_type=F32)
    h = _ln_in(t + b_ref[...] + x_ref[...], g_ref[...], bn_ref[...])
    h_ref[...] = h
    rl = jnp.dot(h, wr_ref[...], preferred_element_type=F32) + br_ref[...]
    rl_ref[...] = rl
    mx = jnp.max(rl, axis=-1, keepdims=True)
    ex = jnp.exp(rl - mx)
    probs = ex / jnp.sum(ex, axis=-1, keepdims=True)
    ent = -jnp.sum(probs * jnp.log(probs + 1e-9))
    ent_ref[...] = ent.reshape(1, 1, 1)
    lane = jax.lax.broadcasted_iota(jnp.int32, (RT, E), 1)
    p1 = jnp.max(probs, axis=-1, keepdims=True)
    i1 = jnp.min(jnp.where(probs == p1, lane, E), axis=-1, keepdims=True)
    pm = jnp.where(lane == i1, -1.0, probs)
    p2 = jnp.max(pm, axis=-1, keepdims=True)
    i2 = jnp.min(jnp.where(pm == p2, lane, E), axis=-1, keepdims=True)
    dn = p1 + p2 + 1e-9
    tv_ref[...] = jnp.concatenate([p1 / dn, p2 / dn], axis=-1)
    ti_ref[...] = jnp.concatenate([i1, i2], axis=-1)


def _proj_router(o, x, w, b, g, bn, wr, br):
    return pl.pallas_call(
        _proj_router_k,
        grid=(NRT,),
        in_specs=[
            pl.BlockSpec((RT, D), lambda i: (i, 0)),
            pl.BlockSpec((RT, D), lambda i: (i, 0)),
            pl.BlockSpec((D, D), lambda i: (0, 0)),
            pl.BlockSpec((1, D), lambda i: (0, 0)),
            pl.BlockSpec((1, D), lambda i: (0, 0)),
            pl.BlockSpec((1, D), lambda i: (0, 0)),
            pl.BlockSpec((D, E), lambda i: (0, 0)),
            pl.BlockSpec((1, E), lambda i: (0, 0)),
        ],
        out_specs=[
            pl.BlockSpec((RT, D), lambda i: (i, 0)),
            pl.BlockSpec((RT, E), lambda i: (i, 0)),
            pl.BlockSpec((RT, 2), lambda i: (i, 0)),
            pl.BlockSpec((RT, 2), lambda i: (i, 0)),
            pl.BlockSpec((1, 1, 1), lambda i: (i, 0, 0)),
        ],
        out_shape=[
            jax.ShapeDtypeStruct((S, D), F32),
            jax.ShapeDtypeStruct((S, E), F32),
            jax.ShapeDtypeStruct((S, 2), F32),
            jax.ShapeDtypeStruct((S, 2), jnp.int32),
            jax.ShapeDtypeStruct((NRT, 1, 1), F32),
        ],
    )(o, x, w, b.reshape(1, D), g.reshape(1, D), bn.reshape(1, D),
      wr, br.reshape(1, E))


def _meta_k(ti_ref, slot_ref, dst_ref, ept_ref, cnt, fill):
    for e in range(E):
        cnt[e] = 0

    def count_body(t, _):
        e0 = ti_ref[2 * t]
        e1 = ti_ref[2 * t + 1]
        cnt[e0] = cnt[e0] + 1
        cnt[e1] = cnt[e1] + 1
        return 0

    jax.lax.fori_loop(0, S, count_body, 0, unroll=4)

    acc = jnp.int32(0)
    offs = []
    for e in range(E):
        offs.append(acc)
        fill[e] = acc
        acc = acc + ((cnt[e] + (M - 1)) // M) * M

    for i in range(NT):
        sel = jnp.int32(0)
        for e in range(1, E):
            sel = jnp.where(offs[e] <= i * M, jnp.int32(e), sel)
        ept_ref[i] = sel

    def fill_body(t, _):
        e0 = ti_ref[2 * t]
        e1 = ti_ref[2 * t + 1]
        d0 = fill[e0]
        fill[e0] = d0 + 1
        d1 = fill[e1]
        fill[e1] = d1 + 1
        dst_ref[2 * t] = d0
        dst_ref[2 * t + 1] = d1
        slot_ref[d0] = t
        slot_ref[d1] = t
        return 0

    jax.lax.fori_loop(0, S, fill_body, 0, unroll=4)


def _route_metadata(ti):
    grid_spec = pltpu.PrefetchScalarGridSpec(
        num_scalar_prefetch=1,
        grid=(1,),
        in_specs=[],
        out_specs=[
            pl.BlockSpec(memory_space=pltpu.SMEM),
            pl.BlockSpec(memory_space=pltpu.SMEM),
            pl.BlockSpec(memory_space=pltpu.SMEM),
        ],
        scratch_shapes=[
            pltpu.SMEM((E,), jnp.int32),
            pltpu.SMEM((E,), jnp.int32),
        ],
    )
    slot, dst, ept = pl.pallas_call(
        _meta_k,
        grid_spec=grid_spec,
        out_shape=[
            jax.ShapeDtypeStruct((NBUF,), jnp.int32),
            jax.ShapeDtypeStruct((2 * S,), jnp.int32),
            jax.ShapeDtypeStruct((NT,), jnp.int32),
        ],
    )(ti.reshape(2 * S))
    return dst.reshape(S, 2), slot, ept


def _moe_k(ept_ref, x_ref, w1_ref, b1_ref, w2_ref, b2_ref, o_ref):
    hid = jnp.dot(x_ref[...], w1_ref[0], preferred_element_type=F32) + b1_ref[0]
    hid = jnp.maximum(hid, 0.0)
    o_ref[...] = jnp.dot(hid.astype(BF16), w2_ref[0],
                         preferred_element_type=F32) + b2_ref[0]


def _moe_grouped(xg, w1, b1, w2, b2, ept):
    grid_spec = pltpu.PrefetchScalarGridSpec(
        num_scalar_prefetch=1,
        grid=(NT,),
        in_specs=[
            pl.BlockSpec((M, D), lambda i, ept: (i, 0)),
            pl.BlockSpec((1, D, DFF), lambda i, ept: (ept[i], 0, 0)),
            pl.BlockSpec((1, 1, DFF), lambda i, ept: (ept[i], 0, 0)),
            pl.BlockSpec((1, DFF, D), lambda i, ept: (ept[i], 0, 0)),
            pl.BlockSpec((1, 1, D), lambda i, ept: (ept[i], 0, 0)),
        ],
        out_specs=pl.BlockSpec((M, D), lambda i, ept: (i, 0)),
    )
    return pl.pallas_call(
        _moe_k,
        grid_spec=grid_spec,
        out_shape=jax.ShapeDtypeStruct((NBUF, D), F32),
    )(ept, xg, w1, b1.reshape(E, 1, DFF), w2, b2.reshape(E, 1, D))


def _combine_k(h_ref, e0_ref, e1_ref, t0_ref, t1_ref, g_ref, b_ref, x_ref):
    moe = t0_ref[...] * e0_ref[...] + t1_ref[...] * e1_ref[...]
    x_ref[...] = _ln_in(h_ref[...] + moe, g_ref[...], b_ref[...])


def _combine(h, e0, e1, t0, t1, g, bn):
    return pl.pallas_call(
        _combine_k,
        grid=(NRT,),
        in_specs=[
            pl.BlockSpec((RT, D), lambda i: (i, 0)),
            pl.BlockSpec((RT, D), lambda i: (i, 0)),
            pl.BlockSpec((RT, D), lambda i: (i, 0)),
            pl.BlockSpec((RT, 1), lambda i: (i, 0)),
            pl.BlockSpec((RT, 1), lambda i: (i, 0)),
            pl.BlockSpec((1, D), lambda i: (0, 0)),
            pl.BlockSpec((1, D), lambda i: (0, 0)),
        ],
        out_specs=pl.BlockSpec((RT, D), lambda i: (i, 0)),
        out_shape=jax.ShapeDtypeStruct((S, D), F32),
    )(h, e0, e1, t0, t1, g.reshape(1, D), bn.reshape(1, D))


def _combine_head_k(h_ref, e0_ref, e1_ref, t0_ref, t1_ref, g_ref, b_ref,
                    wa_ref, ba_ref, gh_ref, bh_ref, wb_ref, bb_ref, tok_ref,
                    mt_ref, out_ref):
    moe = t0_ref[...] * e0_ref[...] + t1_ref[...] * e1_ref[...]
    x = _ln_in(h_ref[...] + moe, g_ref[...], b_ref[...])
    y = jnp.dot(x.astype(BF16), wa_ref[...], preferred_element_type=F32) + ba_ref[...]
    y = _ln_in(y, gh_ref[...], bh_ref[...])
    y = jnp.maximum(y, 0.0)
    lg = jnp.dot(y.astype(BF16), wb_ref[...], preferred_element_type=F32) + bb_ref[...]
    oh = (tok_ref[...] == jax.lax.broadcasted_iota(jnp.int32, (RT, V), 1)).astype(F32)
    am = jnp.dot(oh, mt_ref[...], preferred_element_type=F32)
    out_ref[...] = jnp.where(am > 0.5, lg, -60000.0)


def _combine_head(h, e0, e1, t0, t1, g, bn, wa, ba, gh, bh, wb, bb, tok2d, mt):
    D2 = D // 2
    return pl.pallas_call(
        _combine_head_k,
        grid=(NRT,),
        in_specs=[
            pl.BlockSpec((RT, D), lambda i: (i, 0)),
            pl.BlockSpec((RT, D), lambda i: (i, 0)),
            pl.BlockSpec((RT, D), lambda i: (i, 0)),
            pl.BlockSpec((RT, 1), lambda i: (i, 0)),
            pl.BlockSpec((RT, 1), lambda i: (i, 0)),
            pl.BlockSpec((1, D), lambda i: (0, 0)),
            pl.BlockSpec((1, D), lambda i: (0, 0)),
            pl.BlockSpec((D, D2), lambda i: (0, 0)),
            pl.BlockSpec((1, D2), lambda i: (0, 0)),
            pl.BlockSpec((1, D2), lambda i: (0, 0)),
            pl.BlockSpec((1, D2), lambda i: (0, 0)),
            pl.BlockSpec((D2, V), lambda i: (0, 0)),
            pl.BlockSpec((1, V), lambda i: (0, 0)),
            pl.BlockSpec((RT, 1), lambda i: (i, 0)),
            pl.BlockSpec((V, V), lambda i: (0, 0)),
        ],
        out_specs=pl.BlockSpec((RT, V), lambda i: (i, 0)),
        out_shape=jax.ShapeDtypeStruct((S, V), F32),
    )(h, e0, e1, t0, t1, g.reshape(1, D), bn.reshape(1, D), wa,
      ba.reshape(1, D2), gh.reshape(1, D2), bh.reshape(1, D2), wb,
      bb.reshape(1, V), tok2d, mt)


def kernel(tokenizer_encoded_proteins, mask_table, params):
    p = params
    tok = tokenizer_encoded_proteins.reshape(S).astype(jnp.int32)
    tok2d = tok.reshape(S, 1)
    tok_row = tok.reshape(1, S)
    mt_f = mask_table.astype(F32)

    pos = jnp.arange(S, dtype=F32)[:, None]
    i = jnp.arange(D // 2, dtype=F32)[None, :]
    angle = pos / jnp.power(10000.0, (2.0 * i) / D)
    pe = jnp.concatenate([jnp.sin(angle), jnp.cos(angle)], axis=-1)

    x = _embed(tok2d, p['emb'], pe)

    router_list = []
    ent = []
    per_layer = []
    for l in range(L):
        wqkv = jnp.concatenate([p['Wq'][l], p['Wk'][l], p['Wv'][l]],
                               axis=1).astype(BF16)
        bqkv = jnp.concatenate([p['bq'][l], p['bk'][l], p['bv'][l]], axis=0)
        qkv = _mm_bias(x, wqkv, bqkv)
        q = qkv[:, :D].reshape(S, H, DH).transpose(1, 0, 2)
        k = qkv[:, D:2 * D].reshape(S, H, DH).transpose(1, 0, 2)
        v = qkv[:, 2 * D:].reshape(S, H, DH).transpose(1, 0, 2)
        o = _attention(tok_row, q, k, v)
        o = o.transpose(1, 0, 2).reshape(S, D)
        h, rl, tv, ti, eparts = _proj_router(
            o, x, p['Wo'][l].astype(BF16), p['bo'][l], p['g1'][l],
            p['b1n'][l], p['Wr'][l], p['br'][l])
        router_list.append(rl)
        ent.append(jnp.sum(eparts) / S)
        dst, tok_per_slot, ept = _route_metadata(ti)
        xg = h.astype(BF16)[jnp.clip(tok_per_slot, 0, S - 1)]
        eo = _moe_grouped(xg, p['W1e'][l].astype(BF16), p['b1e'][l],
                          p['W2e'][l].astype(BF16), p['b2e'][l], ept)
        e0 = eo[dst[:, 0]]
        e1 = eo[dst[:, 1]]
        if l < L - 1:
            x = _combine(h, e0, e1, tv[:, 0:1], tv[:, 1:2],
                         p['g2'][l], p['b2n'][l])
        else:
            logits = _combine_head(
                h, e0, e1, tv[:, 0:1], tv[:, 1:2], p['g2'][l], p['b2n'][l],
                p['Wa'].astype(BF16), p['ba'], p['gh'], p['bh'],
                p['Wb'].astype(BF16), p['bb'], tok2d, mt_f)

    entropy_loss = (ent[0] + ent[1]) / L
    return (logits.reshape(B, S, V), router_list[0], router_list[1],
            entropy_loss)


# speedup vs baseline: 1.2215x; 1.1699x over previous
import jax
import jax.numpy as jnp
import numpy as np
from jax.experimental import pallas as pl
from jax.experimental.pallas import tpu as pltpu

B, S, D, H, L, DFF, E = 1, 2048, 1024, 16, 2, 2048, 8
DH = D // H
V = 90
PAD_IDX = 2
RT = 256
NRT = S // RT
M = 256
NBUF = 6144
NT = NBUF // M
F32 = jnp.float32
BF16 = jnp.bfloat16


def _ln_in(x, g, b):
    mu = jnp.mean(x, axis=-1, keepdims=True)
    var = jnp.mean((x - mu) ** 2, axis=-1, keepdims=True)
    return (x - mu) / jnp.sqrt(var + 1e-5) * g + b


def _embed_k(tok_ref, emb_ref, pe_ref, out_ref):
    tok = tok_ref[...]
    oh = (tok == jax.lax.broadcasted_iota(jnp.int32, (RT, V), 1)).astype(F32)
    out_ref[...] = jnp.dot(oh, emb_ref[...], preferred_element_type=F32) + pe_ref[...]


def _embed(tok2d, emb, pe):
    return pl.pallas_call(
        _embed_k,
        grid=(NRT,),
        in_specs=[
            pl.BlockSpec((RT, 1), lambda i: (i, 0)),
            pl.BlockSpec((V, D), lambda i: (0, 0)),
            pl.BlockSpec((RT, D), lambda i: (i, 0)),
        ],
        out_specs=pl.BlockSpec((RT, D), lambda i: (i, 0)),
        out_shape=jax.ShapeDtypeStruct((S, D), F32),
    )(tok2d, emb, pe)


def _mm_bias_k(x_ref, w_ref, b_ref, o_ref):
    r = jnp.dot(x_ref[...].astype(BF16), w_ref[...],
                preferred_element_type=F32) + b_ref[...]
    o_ref[...] = r.astype(o_ref.dtype)


def _mm_bias(x, w, b, out_dtype=F32):
    n = w.shape[1]
    return pl.pallas_call(
        _mm_bias_k,
        grid=(NRT,),
        in_specs=[
            pl.BlockSpec((RT, D), lambda i: (i, 0)),
            pl.BlockSpec((D, n), lambda i: (0, 0)),
            pl.BlockSpec((1, n), lambda i: (0, 0)),
        ],
        out_specs=pl.BlockSpec((RT, n), lambda i: (i, 0)),
        out_shape=jax.ShapeDtypeStruct((S, n), out_dtype),
    )(x, w, b.reshape(1, n))


QB = 512


def _attn_k(tok_ref, q_ref, k_ref, v_ref, o_ref):
    k = k_ref[0]
    ve = v_ref[0]
    pad = tok_ref[...] == PAD_IDX
    for j in range(S // QB):
        q = q_ref[0, j * QB:(j + 1) * QB, :]
        s = jax.lax.dot_general(q, k, (((1,), (1,)), ((), ())),
                                preferred_element_type=F32) * (1.0 / 8.0)
        s = jnp.where(pad, -1e9, s)
        m = jnp.max(s, axis=-1, keepdims=True)
        p = jnp.exp(s - m).astype(BF16)
        oe = jnp.dot(p, ve, preferred_element_type=F32)
        o_ref[0, j * QB:(j + 1) * QB, :] = (
            oe[:, :DH] * (1.0 / oe[:, DH:DH + 1]))


def _attention(tok_row, q, k, ve):
    return pl.pallas_call(
        _attn_k,
        grid=(H,),
        in_specs=[
            pl.BlockSpec((1, S), lambda h: (0, 0)),
            pl.BlockSpec((1, S, DH), lambda h: (h, 0, 0)),
            pl.BlockSpec((1, S, DH), lambda h: (h, 0, 0)),
            pl.BlockSpec((1, S, 2 * DH), lambda h: (h, 0, 0)),
        ],
        out_specs=pl.BlockSpec((1, S, DH), lambda h: (h, 0, 0)),
        out_shape=jax.ShapeDtypeStruct((H, S, DH), F32),
    )(tok_row, q, k, ve)


def _proj_router_k(o_ref, x_ref, w_ref, b_ref, g_ref, bn_ref, wr_ref, br_ref,
                   h_ref, rl_ref, tv_ref, ti_ref, ent_ref):
    t = jnp.dot(o_ref[...].astype(BF16), w_ref[...], preferred_element_type=F32)
    h = _ln_in(t + b_ref[...] + x_ref[...], g_ref[...], bn_ref[...])
    h_ref[...] = h
    rl = jnp.dot(h, wr_ref[...], preferred_element_type=F32) + br_ref[...]
    rl_ref[...] = rl
    mx = jnp.max(rl, axis=-1, keepdims=True)
    ex = jnp.exp(rl - mx)
    probs = ex / jnp.sum(ex, axis=-1, keepdims=True)
    ent = -jnp.sum(probs * jnp.log(probs + 1e-9))
    ent_ref[...] = ent.reshape(1, 1, 1)
    lane = jax.lax.broadcasted_iota(jnp.int32, (RT, E), 1)
    p1 = jnp.max(probs, axis=-1, keepdims=True)
    i1 = jnp.min(jnp.where(probs == p1, lane, E), axis=-1, keepdims=True)
    pm = jnp.where(lane == i1, -1.0, probs)
    p2 = jnp.max(pm, axis=-1, keepdims=True)
    i2 = jnp.min(jnp.where(pm == p2, lane, E), axis=-1, keepdims=True)
    dn = p1 + p2 + 1e-9
    tv_ref[...] = jnp.concatenate([p1 / dn, p2 / dn], axis=-1)
    ti_ref[...] = jnp.concatenate([i1, i2], axis=-1)


def _proj_router(o, x, w, b, g, bn, wr, br):
    return pl.pallas_call(
        _proj_router_k,
        grid=(NRT,),
        in_specs=[
            pl.BlockSpec((RT, D), lambda i: (i, 0)),
            pl.BlockSpec((RT, D), lambda i: (i, 0)),
            pl.BlockSpec((D, D), lambda i: (0, 0)),
            pl.BlockSpec((1, D), lambda i: (0, 0)),
            pl.BlockSpec((1, D), lambda i: (0, 0)),
            pl.BlockSpec((1, D), lambda i: (0, 0)),
            pl.BlockSpec((D, E), lambda i: (0, 0)),
            pl.BlockSpec((1, E), lambda i: (0, 0)),
        ],
        out_specs=[
            pl.BlockSpec((RT, D), lambda i: (i, 0)),
            pl.BlockSpec((RT, E), lambda i: (i, 0)),
            pl.BlockSpec((RT, 2), lambda i: (i, 0)),
            pl.BlockSpec((RT, 2), lambda i: (i, 0)),
            pl.BlockSpec((1, 1, 1), lambda i: (i, 0, 0)),
        ],
        out_shape=[
            jax.ShapeDtypeStruct((S, D), F32),
            jax.ShapeDtypeStruct((S, E), F32),
            jax.ShapeDtypeStruct((S, 2), F32),
            jax.ShapeDtypeStruct((S, 2), jnp.int32),
            jax.ShapeDtypeStruct((NRT, 1, 1), F32),
        ],
    )(o, x, w, b.reshape(1, D), g.reshape(1, D), bn.reshape(1, D),
      wr, br.reshape(1, E))


def _meta_k(ti_ref, slot_ref, dst_ref, ept_ref, cnt, fill):
    for e in range(E):
        cnt[e] = 0

    def count_body(t, _):
        e0 = ti_ref[2 * t]
        e1 = ti_ref[2 * t + 1]
        cnt[e0] = cnt[e0] + 1
        cnt[e1] = cnt[e1] + 1
        return 0

    jax.lax.fori_loop(0, S, count_body, 0, unroll=4)

    acc = jnp.int32(0)
    offs = []
    for e in range(E):
        offs.append(acc)
        fill[e] = acc
        acc = acc + ((cnt[e] + (M - 1)) // M) * M

    for i in range(NT):
        sel = jnp.int32(0)
        for e in range(1, E):
            sel = jnp.where(offs[e] <= i * M, jnp.int32(e), sel)
        ept_ref[i] = sel

    def fill_body(t, _):
        e0 = ti_ref[2 * t]
        e1 = ti_ref[2 * t + 1]
        d0 = fill[e0]
        fill[e0] = d0 + 1
        d1 = fill[e1]
        fill[e1] = d1 + 1
        dst_ref[2 * t] = d0
        dst_ref[2 * t + 1] = d1
        slot_ref[d0] = t
        slot_ref[d1] = t
        return 0

    jax.lax.fori_loop(0, S, fill_body, 0, unroll=4)


def _route_metadata(ti):
    grid_spec = pltpu.PrefetchScalarGridSpec(
        num_scalar_prefetch=1,
        grid=(1,),
        in_specs=[],
        out_specs=[
            pl.BlockSpec(memory_space=pltpu.SMEM),
            pl.BlockSpec(memory_space=pltpu.SMEM),
            pl.BlockSpec(memory_space=pltpu.SMEM),
        ],
        scratch_shapes=[
            pltpu.SMEM((E,), jnp.int32),
            pltpu.SMEM((E,), jnp.int32),
        ],
    )
    slot, dst, ept = pl.pallas_call(
        _meta_k,
        grid_spec=grid_spec,
        out_shape=[
            jax.ShapeDtypeStruct((NBUF,), jnp.int32),
            jax.ShapeDtypeStruct((2 * S,), jnp.int32),
            jax.ShapeDtypeStruct((NT,), jnp.int32),
        ],
    )(ti.reshape(2 * S))
    return dst.reshape(S, 2), slot, ept


def _moe_k(ept_ref, x_ref, w1_ref, b1_ref, w2_ref, b2_ref, o_ref):
    hid = jnp.dot(x_ref[...], w1_ref[0], preferred_element_type=F32) + b1_ref[0]
    hid = jnp.maximum(hid, 0.0)
    o_ref[...] = jnp.dot(hid.astype(BF16), w2_ref[0],
                         preferred_element_type=F32) + b2_ref[0]


def _moe_grouped(xg, w1, b1, w2, b2, ept):
    grid_spec = pltpu.PrefetchScalarGridSpec(
        num_scalar_prefetch=1,
        grid=(NT,),
        in_specs=[
            pl.BlockSpec((M, D), lambda i, ept: (i, 0)),
            pl.BlockSpec((1, D, DFF), lambda i, ept: (ept[i], 0, 0)),
            pl.BlockSpec((1, 1, DFF), lambda i, ept: (ept[i], 0, 0)),
            pl.BlockSpec((1, DFF, D), lambda i, ept: (ept[i], 0, 0)),
            pl.BlockSpec((1, 1, D), lambda i, ept: (ept[i], 0, 0)),
        ],
        out_specs=pl.BlockSpec((M, D), lambda i, ept: (i, 0)),
    )
    return pl.pallas_call(
        _moe_k,
        grid_spec=grid_spec,
        out_shape=jax.ShapeDtypeStruct((NBUF, D), F32),
    )(ept, xg, w1, b1.reshape(E, 1, DFF), w2, b2.reshape(E, 1, D))


def _combine_k(h_ref, e0_ref, e1_ref, t0_ref, t1_ref, g_ref, b_ref, x_ref):
    moe = t0_ref[...] * e0_ref[...] + t1_ref[...] * e1_ref[...]
    x_ref[...] = _ln_in(h_ref[...] + moe, g_ref[...], b_ref[...])


def _combine(h, e0, e1, t0, t1, g, bn):
    return pl.pallas_call(
        _combine_k,
        grid=(NRT,),
        in_specs=[
            pl.BlockSpec((RT, D), lambda i: (i, 0)),
            pl.BlockSpec((RT, D), lambda i: (i, 0)),
            pl.BlockSpec((RT, D), lambda i: (i, 0)),
            pl.BlockSpec((RT, 1), lambda i: (i, 0)),
            pl.BlockSpec((RT, 1), lambda i: (i, 0)),
            pl.BlockSpec((1, D), lambda i: (0, 0)),
            pl.BlockSpec((1, D), lambda i: (0, 0)),
        ],
        out_specs=pl.BlockSpec((RT, D), lambda i: (i, 0)),
        out_shape=jax.ShapeDtypeStruct((S, D), F32),
    )(h, e0, e1, t0, t1, g.reshape(1, D), bn.reshape(1, D))


def _combine_head_k(h_ref, e0_ref, e1_ref, t0_ref, t1_ref, g_ref, b_ref,
                    wa_ref, ba_ref, gh_ref, bh_ref, wb_ref, bb_ref, tok_ref,
                    mt_ref, out_ref):
    moe = t0_ref[...] * e0_ref[...] + t1_ref[...] * e1_ref[...]
    x = _ln_in(h_ref[...] + moe, g_ref[...], b_ref[...])
    y = jnp.dot(x.astype(BF16), wa_ref[...], preferred_element_type=F32) + ba_ref[...]
    y = _ln_in(y, gh_ref[...], bh_ref[...])
    y = jnp.maximum(y, 0.0)
    lg = jnp.dot(y.astype(BF16), wb_ref[...], preferred_element_type=F32) + bb_ref[...]
    oh = (tok_ref[...] == jax.lax.broadcasted_iota(jnp.int32, (RT, V), 1)).astype(F32)
    am = jnp.dot(oh, mt_ref[...], preferred_element_type=F32)
    out_ref[...] = jnp.where(am > 0.5, lg, -60000.0)


def _combine_head(h, e0, e1, t0, t1, g, bn, wa, ba, gh, bh, wb, bb, tok2d, mt):
    D2 = D // 2
    return pl.pallas_call(
        _combine_head_k,
        grid=(NRT,),
        in_specs=[
            pl.BlockSpec((RT, D), lambda i: (i, 0)),
            pl.BlockSpec((RT, D), lambda i: (i, 0)),
            pl.BlockSpec((RT, D), lambda i: (i, 0)),
            pl.BlockSpec((RT, 1), lambda i: (i, 0)),
            pl.BlockSpec((RT, 1), lambda i: (i, 0)),
            pl.BlockSpec((1, D), lambda i: (0, 0)),
            pl.BlockSpec((1, D), lambda i: (0, 0)),
            pl.BlockSpec((D, D2), lambda i: (0, 0)),
            pl.BlockSpec((1, D2), lambda i: (0, 0)),
            pl.BlockSpec((1, D2), lambda i: (0, 0)),
            pl.BlockSpec((1, D2), lambda i: (0, 0)),
            pl.BlockSpec((D2, V), lambda i: (0, 0)),
            pl.BlockSpec((1, V), lambda i: (0, 0)),
            pl.BlockSpec((RT, 1), lambda i: (i, 0)),
            pl.BlockSpec((V, V), lambda i: (0, 0)),
        ],
        out_specs=pl.BlockSpec((RT, V), lambda i: (i, 0)),
        out_shape=jax.ShapeDtypeStruct((S, V), F32),
    )(h, e0, e1, t0, t1, g.reshape(1, D), bn.reshape(1, D), wa,
      ba.reshape(1, D2), gh.reshape(1, D2), bh.reshape(1, D2), wb,
      bb.reshape(1, V), tok2d, mt)


def kernel(tokenizer_encoded_proteins, mask_table, params):
    p = params
    tok = tokenizer_encoded_proteins.reshape(S).astype(jnp.int32)
    tok2d = tok.reshape(S, 1)
    tok_row = tok.reshape(1, S)
    mt_f = mask_table.astype(F32)

    pos = jnp.arange(S, dtype=F32)[:, None]
    i = jnp.arange(D // 2, dtype=F32)[None, :]
    angle = pos / jnp.power(10000.0, (2.0 * i) / D)
    pe = jnp.concatenate([jnp.sin(angle), jnp.cos(angle)], axis=-1)

    x = _embed(tok2d, p['emb'], pe)

    router_list = []
    ent = []
    per_layer = []
    for l in range(L):
        wqkv = jnp.concatenate([p['Wq'][l], p['Wk'][l], p['Wv'][l]],
                               axis=1).astype(BF16)
        bqkv = jnp.concatenate([p['bq'][l], p['bk'][l], p['bv'][l]], axis=0)
        qkv = _mm_bias(x, wqkv, bqkv, out_dtype=BF16)
        q = qkv[:, :D].reshape(S, H, DH).transpose(1, 0, 2)
        k = qkv[:, D:2 * D].reshape(S, H, DH).transpose(1, 0, 2)
        v = qkv[:, 2 * D:].reshape(S, H, DH).transpose(1, 0, 2)
        ones = jnp.ones((H, S, 1), BF16)
        ve = jnp.concatenate([v, ones, jnp.zeros((H, S, DH - 1), BF16)],
                             axis=-1)
        o = _attention(tok_row, q, k, ve)
        o = o.transpose(1, 0, 2).reshape(S, D)
        h, rl, tv, ti, eparts = _proj_router(
            o, x, p['Wo'][l].astype(BF16), p['bo'][l], p['g1'][l],
            p['b1n'][l], p['Wr'][l], p['br'][l])
        router_list.append(rl)
        ent.append(jnp.sum(eparts) / S)
        dst, tok_per_slot, ept = _route_metadata(ti)
        xg = h.astype(BF16)[jnp.clip(tok_per_slot, 0, S - 1)]
        eo = _moe_grouped(xg, p['W1e'][l].astype(BF16), p['b1e'][l],
                          p['W2e'][l].astype(BF16), p['b2e'][l], ept)
        e0 = eo[dst[:, 0]]
        e1 = eo[dst[:, 1]]
        if l < L - 1:
            x = _combine(h, e0, e1, tv[:, 0:1], tv[:, 1:2],
                         p['g2'][l], p['b2n'][l])
        else:
            logits = _combine_head(
                h, e0, e1, tv[:, 0:1], tv[:, 1:2], p['g2'][l], p['b2n'][l],
                p['Wa'].astype(BF16), p['ba'], p['gh'], p['bh'],
                p['Wb'].astype(BF16), p['bb'], tok2d, mt_f)

    entropy_loss = (ent[0] + ent[1]) / L
    return (logits.reshape(B, S, V), router_list[0], router_list[1],
            entropy_loss)


# speedup vs baseline: 1.2586x; 1.0303x over previous
import jax
import jax.numpy as jnp
import numpy as np
from jax.experimental import pallas as pl
from jax.experimental.pallas import tpu as pltpu

B, S, D, H, L, DFF, E = 1, 2048, 1024, 16, 2, 2048, 8
DH = D // H
V = 90
PAD_IDX = 2
RT = 256
NRT = S // RT
M = 256
NBUF = 6144
NT = NBUF // M
F32 = jnp.float32
BF16 = jnp.bfloat16


def _ln_in(x, g, b):
    mu = jnp.mean(x, axis=-1, keepdims=True)
    var = jnp.mean((x - mu) ** 2, axis=-1, keepdims=True)
    return (x - mu) / jnp.sqrt(var + 1e-5) * g + b


def _embed_k(tok_ref, emb_ref, pe_ref, out_ref):
    tok = tok_ref[...]
    oh = (tok == jax.lax.broadcasted_iota(jnp.int32, (RT, V), 1)).astype(F32)
    out_ref[...] = jnp.dot(oh, emb_ref[...], preferred_element_type=F32) + pe_ref[...]


def _embed(tok2d, emb, pe):
    return pl.pallas_call(
        _embed_k,
        grid=(NRT,),
        in_specs=[
            pl.BlockSpec((RT, 1), lambda i: (i, 0)),
            pl.BlockSpec((V, D), lambda i: (0, 0)),
            pl.BlockSpec((RT, D), lambda i: (i, 0)),
        ],
        out_specs=pl.BlockSpec((RT, D), lambda i: (i, 0)),
        out_shape=jax.ShapeDtypeStruct((S, D), F32),
    )(tok2d, emb, pe)


def _mm_bias_k(x_ref, w_ref, b_ref, o_ref):
    r = jnp.dot(x_ref[...].astype(BF16), w_ref[...],
                preferred_element_type=F32) + b_ref[...]
    o_ref[...] = r.astype(o_ref.dtype)


def _mm_bias(x, w, b, out_dtype=F32):
    n = w.shape[1]
    return pl.pallas_call(
        _mm_bias_k,
        grid=(NRT,),
        in_specs=[
            pl.BlockSpec((RT, D), lambda i: (i, 0)),
            pl.BlockSpec((D, n), lambda i: (0, 0)),
            pl.BlockSpec((1, n), lambda i: (0, 0)),
        ],
        out_specs=pl.BlockSpec((RT, n), lambda i: (i, 0)),
        out_shape=jax.ShapeDtypeStruct((S, n), out_dtype),
    )(x, w, b.reshape(1, n))


QB = 512


def _attn_k(tok_ref, q_ref, k_ref, v_ref, o_ref):
    k = k_ref[0]
    ve = v_ref[0]
    pad = tok_ref[...] == PAD_IDX
    for j in range(S // QB):
        q = q_ref[0, j * QB:(j + 1) * QB, :]
        s = jax.lax.dot_general(q, k, (((1,), (1,)), ((), ())),
                                preferred_element_type=F32) * (1.0 / 8.0)
        s = jnp.where(pad, -1e9, s)
        m = jnp.max(s, axis=-1, keepdims=True)
        p = jnp.exp(s - m).astype(BF16)
        oe = jnp.dot(p, ve, preferred_element_type=F32)
        o_ref[0, j * QB:(j + 1) * QB, :] = (
            oe[:, :DH] * (1.0 / oe[:, DH:DH + 1]))


def _attention(tok_row, q, k, ve):
    return pl.pallas_call(
        _attn_k,
        grid=(H,),
        in_specs=[
            pl.BlockSpec((1, S), lambda h: (0, 0)),
            pl.BlockSpec((1, S, DH), lambda h: (h, 0, 0)),
            pl.BlockSpec((1, S, DH), lambda h: (h, 0, 0)),
            pl.BlockSpec((1, S, 2 * DH), lambda h: (h, 0, 0)),
        ],
        out_specs=pl.BlockSpec((1, S, DH), lambda h: (h, 0, 0)),
        out_shape=jax.ShapeDtypeStruct((H, S, DH), F32),
    )(tok_row, q, k, ve)


def _proj_router_k(o_ref, x_ref, w_ref, b_ref, g_ref, bn_ref, wr_ref, br_ref,
                   h_ref, rl_ref, tv_ref, ti_ref, pos_ref, ent_ref, runc_ref,
                   carry):
    t = jnp.dot(o_ref[...].astype(BF16), w_ref[...], preferred_element_type=F32)
    h = _ln_in(t + b_ref[...] + x_ref[...], g_ref[...], bn_ref[...])
    h_ref[...] = h
    rl = jnp.dot(h, wr_ref[...], preferred_element_type=F32) + br_ref[...]
    rl_ref[...] = rl
    mx = jnp.max(rl, axis=-1, keepdims=True)
    ex = jnp.exp(rl - mx)
    probs = ex / jnp.sum(ex, axis=-1, keepdims=True)
    ent = -jnp.sum(probs * jnp.log(probs + 1e-9))
    ent_ref[...] = ent.reshape(1, 1, 1)
    lane = jax.lax.broadcasted_iota(jnp.int32, (RT, E), 1)
    p1 = jnp.max(probs, axis=-1, keepdims=True)
    i1 = jnp.min(jnp.where(probs == p1, lane, E), axis=-1, keepdims=True)
    pm = jnp.where(lane == i1, -1.0, probs)
    p2 = jnp.max(pm, axis=-1, keepdims=True)
    i2 = jnp.min(jnp.where(pm == p2, lane, E), axis=-1, keepdims=True)
    dn = p1 + p2 + 1e-9
    tv_ref[...] = jnp.concatenate([p1 / dn, p2 / dn], axis=-1)
    ti_ref[...] = jnp.concatenate([i1, i2], axis=-1)

    @pl.when(pl.program_id(0) == 0)
    def _():
        carry[...] = jnp.zeros((1, E), jnp.int32)

    oh0 = (lane == i1).astype(jnp.int32)
    oh1 = (lane == i2).astype(jnp.int32)
    flags = oh0 + oh1
    c = flags
    k = 1
    while k < RT:
        sh = jnp.concatenate(
            [jnp.zeros((k, E), jnp.int32), c[:RT - k]], axis=0)
        c = c + sh
        k *= 2
    excl = c - flags + carry[...]
    pos0 = jnp.sum(excl * oh0, axis=-1, keepdims=True)
    pos1 = jnp.sum(excl * oh1, axis=-1, keepdims=True)
    pos_ref[...] = jnp.concatenate([pos0, pos1], axis=-1)
    carry[...] = carry[...] + c[RT - 1:RT, :]
    runc_ref[...] = carry[...].reshape(1, 1, E)


def _proj_router(o, x, w, b, g, bn, wr, br):
    return pl.pallas_call(
        _proj_router_k,
        grid=(NRT,),
        in_specs=[
            pl.BlockSpec((RT, D), lambda i: (i, 0)),
            pl.BlockSpec((RT, D), lambda i: (i, 0)),
            pl.BlockSpec((D, D), lambda i: (0, 0)),
            pl.BlockSpec((1, D), lambda i: (0, 0)),
            pl.BlockSpec((1, D), lambda i: (0, 0)),
            pl.BlockSpec((1, D), lambda i: (0, 0)),
            pl.BlockSpec((D, E), lambda i: (0, 0)),
            pl.BlockSpec((1, E), lambda i: (0, 0)),
        ],
        out_specs=[
            pl.BlockSpec((RT, D), lambda i: (i, 0)),
            pl.BlockSpec((RT, E), lambda i: (i, 0)),
            pl.BlockSpec((RT, 2), lambda i: (i, 0)),
            pl.BlockSpec((RT, 2), lambda i: (i, 0)),
            pl.BlockSpec((RT, 2), lambda i: (i, 0)),
            pl.BlockSpec((1, 1, 1), lambda i: (i, 0, 0)),
            pl.BlockSpec((1, 1, E), lambda i: (i, 0, 0)),
        ],
        out_shape=[
            jax.ShapeDtypeStruct((S, D), F32),
            jax.ShapeDtypeStruct((S, E), F32),
            jax.ShapeDtypeStruct((S, 2), F32),
            jax.ShapeDtypeStruct((S, 2), jnp.int32),
            jax.ShapeDtypeStruct((S, 2), jnp.int32),
            jax.ShapeDtypeStruct((NRT, 1, 1), F32),
            jax.ShapeDtypeStruct((NRT, 1, E), jnp.int32),
        ],
        scratch_shapes=[pltpu.VMEM((1, E), jnp.int32)],
    )(o, x, w, b.reshape(1, D), g.reshape(1, D), bn.reshape(1, D),
      wr, br.reshape(1, E))


def _route_metadata(ti, pos, counts):
    ru = ((counts + M - 1) // M) * M
    off = jnp.concatenate([jnp.zeros((1,), jnp.int32),
                           jnp.cumsum(ru)[:-1].astype(jnp.int32)])
    dst = off[ti] + pos
    tokid = jnp.broadcast_to(jnp.arange(S, dtype=jnp.int32)[:, None], (S, 2))
    tok_per_slot = jnp.zeros((NBUF,), jnp.int32).at[dst.reshape(-1)].set(
        tokid.reshape(-1), mode="drop")
    tile_starts = jnp.arange(NT, dtype=jnp.int32) * M
    ept = (jnp.searchsorted(off, tile_starts, side="right") - 1).astype(jnp.int32)
    return dst, tok_per_slot, ept


def _moe_k(ept_ref, x_ref, w1_ref, b1_ref, w2_ref, b2_ref, o_ref):
    hid = jnp.dot(x_ref[...], w1_ref[0], preferred_element_type=F32) + b1_ref[0]
    hid = jnp.maximum(hid, 0.0)
    o_ref[...] = jnp.dot(hid.astype(BF16), w2_ref[0],
                         preferred_element_type=F32) + b2_ref[0]


def _moe_grouped(xg, w1, b1, w2, b2, ept):
    grid_spec = pltpu.PrefetchScalarGridSpec(
        num_scalar_prefetch=1,
        grid=(NT,),
        in_specs=[
            pl.BlockSpec((M, D), lambda i, ept: (i, 0)),
            pl.BlockSpec((1, D, DFF), lambda i, ept: (ept[i], 0, 0)),
            pl.BlockSpec((1, 1, DFF), lambda i, ept: (ept[i], 0, 0)),
            pl.BlockSpec((1, DFF, D), lambda i, ept: (ept[i], 0, 0)),
            pl.BlockSpec((1, 1, D), lambda i, ept: (ept[i], 0, 0)),
        ],
        out_specs=pl.BlockSpec((M, D), lambda i, ept: (i, 0)),
    )
    return pl.pallas_call(
        _moe_k,
        grid_spec=grid_spec,
        out_shape=jax.ShapeDtypeStruct((NBUF, D), F32),
    )(ept, xg, w1, b1.reshape(E, 1, DFF), w2, b2.reshape(E, 1, D))


def _combine_k(h_ref, e0_ref, e1_ref, t0_ref, t1_ref, g_ref, b_ref, x_ref):
    moe = t0_ref[...] * e0_ref[...] + t1_ref[...] * e1_ref[...]
    x_ref[...] = _ln_in(h_ref[...] + moe, g_ref[...], b_ref[...])


def _combine(h, e0, e1, t0, t1, g, bn):
    return pl.pallas_call(
        _combine_k,
        grid=(NRT,),
        in_specs=[
            pl.BlockSpec((RT, D), lambda i: (i, 0)),
            pl.BlockSpec((RT, D), lambda i: (i, 0)),
            pl.BlockSpec((RT, D), lambda i: (i, 0)),
            pl.BlockSpec((RT, 1), lambda i: (i, 0)),
            pl.BlockSpec((RT, 1), lambda i: (i, 0)),
            pl.BlockSpec((1, D), lambda i: (0, 0)),
            pl.BlockSpec((1, D), lambda i: (0, 0)),
        ],
        out_specs=pl.BlockSpec((RT, D), lambda i: (i, 0)),
        out_shape=jax.ShapeDtypeStruct((S, D), F32),
    )(h, e0, e1, t0, t1, g.reshape(1, D), bn.reshape(1, D))


def _combine_head_k(h_ref, e0_ref, e1_ref, t0_ref, t1_ref, g_ref, b_ref,
                    wa_ref, ba_ref, gh_ref, bh_ref, wb_ref, bb_ref, tok_ref,
                    mt_ref, out_ref):
    moe = t0_ref[...] * e0_ref[...] + t1_ref[...] * e1_ref[...]
    x = _ln_in(h_ref[...] + moe, g_ref[...], b_ref[...])
    y = jnp.dot(x.astype(BF16), wa_ref[...], preferred_element_type=F32) + ba_ref[...]
    y = _ln_in(y, gh_ref[...], bh_ref[...])
    y = jnp.maximum(y, 0.0)
    lg = jnp.dot(y.astype(BF16), wb_ref[...], preferred_element_type=F32) + bb_ref[...]
    oh = (tok_ref[...] == jax.lax.broadcasted_iota(jnp.int32, (RT, V), 1)).astype(F32)
    am = jnp.dot(oh, mt_ref[...], preferred_element_type=F32)
    out_ref[...] = jnp.where(am > 0.5, lg, -60000.0)


def _combine_head(h, e0, e1, t0, t1, g, bn, wa, ba, gh, bh, wb, bb, tok2d, mt):
    D2 = D // 2
    return pl.pallas_call(
        _combine_head_k,
        grid=(NRT,),
        in_specs=[
            pl.BlockSpec((RT, D), lambda i: (i, 0)),
            pl.BlockSpec((RT, D), lambda i: (i, 0)),
            pl.BlockSpec((RT, D), lambda i: (i, 0)),
            pl.BlockSpec((RT, 1), lambda i: (i, 0)),
            pl.BlockSpec((RT, 1), lambda i: (i, 0)),
            pl.BlockSpec((1, D), lambda i: (0, 0)),
            pl.BlockSpec((1, D), lambda i: (0, 0)),
            pl.BlockSpec((D, D2), lambda i: (0, 0)),
            pl.BlockSpec((1, D2), lambda i: (0, 0)),
            pl.BlockSpec((1, D2), lambda i: (0, 0)),
            pl.BlockSpec((1, D2), lambda i: (0, 0)),
            pl.BlockSpec((D2, V), lambda i: (0, 0)),
            pl.BlockSpec((1, V), lambda i: (0, 0)),
            pl.BlockSpec((RT, 1), lambda i: (i, 0)),
            pl.BlockSpec((V, V), lambda i: (0, 0)),
        ],
        out_specs=pl.BlockSpec((RT, V), lambda i: (i, 0)),
        out_shape=jax.ShapeDtypeStruct((S, V), F32),
    )(h, e0, e1, t0, t1, g.reshape(1, D), bn.reshape(1, D), wa,
      ba.reshape(1, D2), gh.reshape(1, D2), bh.reshape(1, D2), wb,
      bb.reshape(1, V), tok2d, mt)


def kernel(tokenizer_encoded_proteins, mask_table, params):
    p = params
    tok = tokenizer_encoded_proteins.reshape(S).astype(jnp.int32)
    tok2d = tok.reshape(S, 1)
    tok_row = tok.reshape(1, S)
    mt_f = mask_table.astype(F32)

    pos = jnp.arange(S, dtype=F32)[:, None]
    i = jnp.arange(D // 2, dtype=F32)[None, :]
    angle = pos / jnp.power(10000.0, (2.0 * i) / D)
    pe = jnp.concatenate([jnp.sin(angle), jnp.cos(angle)], axis=-1)

    x = _embed(tok2d, p['emb'], pe)

    router_list = []
    ent = []
    per_layer = []
    for l in range(L):
        wqkv = jnp.concatenate([p['Wq'][l], p['Wk'][l], p['Wv'][l]],
                               axis=1).astype(BF16)
        bqkv = jnp.concatenate([p['bq'][l], p['bk'][l], p['bv'][l]], axis=0)
        qkv = _mm_bias(x, wqkv, bqkv, out_dtype=BF16)
        q = qkv[:, :D].reshape(S, H, DH).transpose(1, 0, 2)
        k = qkv[:, D:2 * D].reshape(S, H, DH).transpose(1, 0, 2)
        v = qkv[:, 2 * D:].reshape(S, H, DH).transpose(1, 0, 2)
        ones = jnp.ones((H, S, 1), BF16)
        ve = jnp.concatenate([v, ones, jnp.zeros((H, S, DH - 1), BF16)],
                             axis=-1)
        o = _attention(tok_row, q, k, ve)
        o = o.transpose(1, 0, 2).reshape(S, D)
        h, rl, tv, ti, pos, eparts, runc = _proj_router(
            o, x, p['Wo'][l].astype(BF16), p['bo'][l], p['g1'][l],
            p['b1n'][l], p['Wr'][l], p['br'][l])
        router_list.append(rl)
        ent.append(jnp.sum(eparts) / S)
        counts = runc[NRT - 1, 0]
        dst, tok_per_slot, ept = _route_metadata(ti, pos, counts)
        xg = h.astype(BF16)[tok_per_slot]
        eo = _moe_grouped(xg, p['W1e'][l].astype(BF16), p['b1e'][l],
                          p['W2e'][l].astype(BF16), p['b2e'][l], ept)
        e0 = eo[dst[:, 0]]
        e1 = eo[dst[:, 1]]
        if l < L - 1:
            x = _combine(h, e0, e1, tv[:, 0:1], tv[:, 1:2],
                         p['g2'][l], p['b2n'][l])
        else:
            logits = _combine_head(
                h, e0, e1, tv[:, 0:1], tv[:, 1:2], p['g2'][l], p['b2n'][l],
                p['Wa'].astype(BF16), p['ba'], p['gh'], p['bh'],
                p['Wb'].astype(BF16), p['bb'], tok2d, mt_f)

    entropy_loss = (ent[0] + ent[1]) / L
    return (logits.reshape(B, S, V), router_list[0], router_list[1],
            entropy_loss)
